# baseline (speedup 1.0000x reference)
import jax
import jax.numpy as jnp
from jax import lax
from jax.experimental import pallas as pl
from jax.experimental.pallas import tpu as pltpu
from jax.experimental.pallas import tpu_sc as plsc

N = 10000
DIM = 128
E = 320000
TWO_E = 2 * E
NC, NS, L = 2, 16, 16
CHUNK = 128
BLK = 32
NCH_W = 320
EPAD = NS * NCH_W * CHUNK
NROW = EPAD // CHUNK
NACC = N + 8
DEG_W = TWO_E // NS
SPAD = 4608
SROW = SPAD // CHUNK
SEEDS = 4500

_mesh = plsc.VectorSubcoreMesh(
    core_axis_name="c", subcore_axis_name="s", num_cores=NC, num_subcores=NS)
_sc_params = pltpu.CompilerParams(needs_layout_passes=False)


def _deg_body(edges, out, idx_v, hist_v):
    c = lax.axis_index("c")
    s = lax.axis_index("s")

    zeros = jnp.zeros((L,), jnp.float32)

    def zero_body(i, _):
        hist_v[pl.ds(i * L, L)] = zeros
        return 0

    lax.fori_loop(0, N // L, zero_body, 0)

    pltpu.sync_copy(edges.at[pl.ds(c * TWO_E + s * DEG_W, DEG_W)], idx_v)

    ones = jnp.ones((L,), jnp.float32)

    def body(i, _):
        v = idx_v[pl.ds(i * L, L)]
        plsc.addupdate_scatter(hist_v, [v], ones)
        return 0

    lax.fori_loop(0, DEG_W // L, body, 0)
    pltpu.sync_copy(hist_v, out.at[c, s])


_deg_call = pl.kernel(
    _deg_body,
    out_type=jax.ShapeDtypeStruct((NC, NS, N), jnp.float32),
    mesh=_mesh,
    scratch_types=[
        pltpu.VMEM((DEG_W,), jnp.int32),
        pltpu.VMEM((N,), jnp.float32),
    ],
    compiler_params=_sc_params,
)


def _prop_body(g, srcs, dsts, out,
               idxs2, idxd2, rows0, rows1, acc_sh, sem0, sem1):
    c = lax.axis_index("c")
    s = lax.axis_index("s")

    t_off = 16 * 624 + 8 * jnp.minimum(s, 1)
    pltpu.sync_copy(g.at[pl.ds(c * N + s * 624, 624)],
                    acc_sh.at[pl.ds(s * 624, 624)])
    pltpu.sync_copy(g.at[pl.ds(c * N + t_off, 8)],
                    acc_sh.at[pl.ds(t_off, 8)])
    plsc.subcore_barrier()

    def wait_rows(buf, sem):
        pltpu.make_async_copy(g.at[pl.ds(0, CHUNK)], buf, sem).wait()

    def blk_body(b, _):
        boff = c * NROW + s * NCH_W + b * BLK
        pltpu.sync_copy(srcs.at[pl.ds(boff, BLK)], idxs2)
        pltpu.sync_copy(dsts.at[pl.ds(boff, BLK)], idxd2)
        pltpu.async_copy(g.at[idxs2.at[0]], rows0, sem0)

        def body(i, _):
            t0 = 2 * i
            pltpu.async_copy(g.at[idxs2.at[t0 + 1]], rows1, sem1)
            wait_rows(rows0, sem0)
            pltpu.sync_copy(rows0, acc_sh.at[idxd2.at[t0]], add=True)
            t2 = jnp.minimum(t0 + 2, BLK - 1)
            pltpu.async_copy(g.at[idxs2.at[t2]], rows0, sem0)
            wait_rows(rows1, sem1)
            pltpu.sync_copy(rows1, acc_sh.at[idxd2.at[t0 + 1]], add=True)
            return 0

        lax.fori_loop(0, BLK // 2, body, 0)
        wait_rows(rows0, sem0)
        return 0

    lax.fori_loop(0, NCH_W // BLK, blk_body, 0)
    plsc.subcore_barrier()
    pltpu.sync_copy(acc_sh.at[pl.ds(s * 624, 624)],
                    out.at[pl.ds(c * N + s * 624, 624)])
    pltpu.sync_copy(acc_sh.at[pl.ds(t_off, 8)],
                    out.at[pl.ds(c * N + t_off, 8)])


_prop_call = pl.kernel(
    _prop_body,
    out_type=jax.ShapeDtypeStruct((NC * N, DIM), jnp.float32),
    mesh=_mesh,
    scratch_types=[
        pltpu.VMEM((BLK, CHUNK), jnp.int32),
        pltpu.VMEM((BLK, CHUNK), jnp.int32),
        pltpu.VMEM((CHUNK, DIM), jnp.float32),
        pltpu.VMEM((CHUNK, DIM), jnp.float32),
        pltpu.VMEM_SHARED((NACC, DIM), jnp.float32),
        pltpu.SemaphoreType.DMA,
        pltpu.SemaphoreType.DMA,
    ],
    compiler_params=_sc_params,
)


def _seed_body(hid, seeds, out, idx_v, rows_v, sem):
    c = lax.axis_index("c")
    s = lax.axis_index("s")

    def do_chunk(j):
        pltpu.sync_copy(seeds.at[pl.ds((c * SROW + j) * CHUNK, CHUNK)], idx_v)
        pltpu.async_copy(hid.at[idx_v], rows_v, sem).wait()
        pltpu.sync_copy(rows_v, out.at[pl.ds((c * SROW + j) * CHUNK, CHUNK)])

    do_chunk(s)
    do_chunk(s + NS)
    do_chunk(jnp.minimum(s + 2 * NS, SROW - 1))


_seed_call = pl.kernel(
    _seed_body,
    out_type=jax.ShapeDtypeStruct((NC * SPAD, DIM), jnp.float32),
    mesh=_mesh,
    scratch_types=[
        pltpu.VMEM((CHUNK,), jnp.int32),
        pltpu.VMEM((CHUNK, DIM), jnp.float32),
        pltpu.SemaphoreType.DMA,
    ],
    compiler_params=_sc_params,
)


def _dense0_body(part_ref, feats_sr_ref, feats_tg_ref,
                 g0_ref, f_sr_ref, f_tg_ref):
    part = part_ref[...]
    f_sr = lax.rsqrt(jnp.sum(part[0], axis=0) + 1.0)
    f_tg = lax.rsqrt(jnp.sum(part[1], axis=0) + 1.0)
    f_sr_ref[...] = f_sr
    f_tg_ref[...] = f_tg
    g0_ref[:N, :] = feats_sr_ref[...] * f_sr[:, None]
    g0_ref[N:, :] = feats_tg_ref[...] * f_tg[:, None]


def _dense0(part, feats_sr, feats_tg):
    return pl.pallas_call(
        _dense0_body,
        out_shape=(jax.ShapeDtypeStruct((NC * N, DIM), jnp.float32),
                   jax.ShapeDtypeStruct((N,), jnp.float32),
                   jax.ShapeDtypeStruct((N,), jnp.float32)),
    )(part, feats_sr, feats_tg)


def _dense1_body(agg_ref, f_sr_ref, f_tg_ref, w_ref, g1_ref):
    w = w_ref[...]

    def one(agg, f):
        h = jnp.dot(agg * f[:, None], w, preferred_element_type=jnp.float32)
        return jnp.maximum(h, 0.0) * f[:, None]

    g1_ref[:N, :] = one(agg_ref[:N, :], f_sr_ref[...])
    g1_ref[N:, :] = one(agg_ref[N:, :], f_tg_ref[...])


def _dense1(agg, f_sr, f_tg, w):
    return pl.pallas_call(
        _dense1_body,
        out_shape=jax.ShapeDtypeStruct((NC * N, DIM), jnp.float32),
    )(agg, f_sr, f_tg, w)


def _dense2_body(agg_ref, f_sr_ref, f_tg_ref, w_ref, hid_ref):
    w = w_ref[...]

    def one(agg, f):
        h = jnp.dot(agg * f[:, None], w, preferred_element_type=jnp.float32)
        nrm = jnp.sqrt(jnp.sum(h * h, axis=-1, keepdims=True))
        return h / jnp.maximum(nrm, 1e-12)

    hid_ref[:N, :] = one(agg_ref[:N, :], f_sr_ref[...])
    hid_ref[N:, :] = one(agg_ref[N:, :], f_tg_ref[...])


def _dense2(agg, f_sr, f_tg, w):
    return pl.pallas_call(
        _dense2_body,
        out_shape=jax.ShapeDtypeStruct((NC * N, DIM), jnp.float32),
    )(agg, f_sr, f_tg, w)


def _edge_lists(edges, node_off):
    pad = EPAD - TWO_E
    spread = jnp.arange(pad, dtype=jnp.int32)
    src = jnp.concatenate(
        [edges[:, 0] + node_off, edges[:, 1] + node_off,
         node_off + (spread % N)])
    dst = jnp.concatenate(
        [edges[:, 1], edges[:, 0], N + (spread % 8)])
    return src.reshape(NROW, CHUNK), dst.reshape(NROW, CHUNK)


def _pad_seeds(seeds, node_off):
    return jnp.concatenate(
        [seeds + node_off, jnp.full((SPAD - SEEDS,), node_off, jnp.int32)])


def kernel(feats_sr, feats_tg, W0, W1, edges_sr, edges_tg,
           sr_ent_seeds, tg_ent_seeds, triples_sr, triples_tg):
    edges_flat = jnp.concatenate(
        [edges_sr.reshape(-1), edges_tg.reshape(-1)])
    part = _deg_call(edges_flat)
    g0, f_sr, f_tg = _dense0(part, feats_sr, feats_tg)

    src_sr, dst_sr = _edge_lists(edges_sr, 0)
    src_tg, dst_tg = _edge_lists(edges_tg, N)
    srcs = jnp.concatenate([src_sr, src_tg])
    dsts = jnp.concatenate([dst_sr, dst_tg])

    agg0 = _prop_call(g0, srcs, dsts)
    g1 = agg0
    agg1 = _prop_call(g1, srcs, dsts)
    hid = _dense2(agg1, f_sr, f_tg, W1)

    seeds = jnp.concatenate(
        [_pad_seeds(sr_ent_seeds, 0), _pad_seeds(tg_ent_seeds, N)])
    seed_out = _seed_call(hid, seeds)
    return (seed_out[:SEEDS], seed_out[SPAD:SPAD + SEEDS],
            hid[:N], hid[N:])


# speedup vs baseline: 1.0030x; 1.0030x over previous
import jax
import jax.numpy as jnp
from jax import lax
from jax.experimental import pallas as pl
from jax.experimental.pallas import tpu as pltpu
from jax.experimental.pallas import tpu_sc as plsc

N = 10000
DIM = 128
E = 320000
NC, NS, L = 2, 16, 16
CHUNK = 128
BLK = 16
NBLK = 10
PAIRS_W = NBLK * BLK * CHUNK
EPP = NS * PAIRS_W
PADP = EPP - E
NSAC = 128
NACC = N + NSAC
NG = NC * N + NSAC
DEG_W = 2 * EPP // NS
SPAD = 4608
SROW = SPAD // CHUNK
SEEDS = 4500

_mesh = plsc.VectorSubcoreMesh(
    core_axis_name="c", subcore_axis_name="s", num_cores=NC, num_subcores=NS)
_sc_params = pltpu.CompilerParams(needs_layout_passes=False)


def _deg_body(pairs, out, idx_v, hist_v):
    c = lax.axis_index("c")
    s = lax.axis_index("s")

    zeros = jnp.zeros((L,), jnp.float32)

    def zero_body(i, _):
        hist_v[pl.ds(i * L, L)] = zeros
        return 0

    lax.fori_loop(0, NACC // L, zero_body, 0)

    pltpu.sync_copy(pairs.at[pl.ds(c * 2 * EPP + s * DEG_W, DEG_W)], idx_v)

    ones = jnp.ones((L,), jnp.float32)

    def body(i, _):
        v = idx_v[pl.ds(i * L, L)]
        plsc.addupdate_scatter(hist_v, [v], ones)
        return 0

    lax.fori_loop(0, DEG_W // L, body, 0)
    pltpu.sync_copy(hist_v, out.at[c, s])


_deg_call = pl.kernel(
    _deg_body,
    out_type=jax.ShapeDtypeStruct((NC, NS, NACC), jnp.float32),
    mesh=_mesh,
    scratch_types=[
        pltpu.VMEM((DEG_W,), jnp.int32),
        pltpu.VMEM((NACC,), jnp.float32),
    ],
    compiler_params=_sc_params,
)


def _prop_body(g, pairs, out,
               pairbuf, idxg2, idxs2, rows0, rows1, acc_sh, sem0, sem1):
    c = lax.axis_index("c")
    s = lax.axis_index("s")
    cN = c * N

    t_off = 16 * 624 + 8 * jnp.minimum(s, 1)
    pltpu.sync_copy(g.at[pl.ds(cN + s * 624, 624)],
                    acc_sh.at[pl.ds(s * 624, 624)])
    pltpu.sync_copy(g.at[pl.ds(cN + t_off, 8)],
                    acc_sh.at[pl.ds(t_off, 8)])
    plsc.subcore_barrier()

    def wait_rows(buf, sem):
        pltpu.make_async_copy(g.at[pl.ds(0, CHUNK)], buf, sem).wait()

    iota16 = lax.iota(jnp.int32, L)
    rowpat = iota16 % 2
    colpat = iota16 // 2

    def blk_body(b, _):
        ioff = c * 2 * EPP + s * 2 * PAIRS_W + b * (BLK * 2 * CHUNK)
        pltpu.sync_copy(pairs.at[pl.ds(ioff, BLK * 2 * CHUNK)], pairbuf)

        def dv(i, _):
            off = i * L
            v = pairbuf[pl.ds(off, L)]
            j = off // (2 * CHUNK)
            p8 = (off % (2 * CHUNK)) // 2
            rows = 2 * j + rowpat
            cols = p8 + colpat
            plsc.store_scatter(idxs2, [rows, cols], v)
            plsc.store_scatter(idxg2, [rows, cols], v + cN)
            return 0

        lax.fori_loop(0, BLK * 2 * CHUNK // L, dv, 0)

        pltpu.async_copy(g.at[idxg2.at[0]], rows0, sem0)

        def body(i, _):
            t0 = 2 * i
            pltpu.async_copy(g.at[idxg2.at[t0 + 1]], rows1, sem1)
            wait_rows(rows0, sem0)
            pltpu.sync_copy(rows0, acc_sh.at[idxs2.at[t0 + 1]], add=True)
            t2 = jnp.minimum(t0 + 2, 2 * BLK - 1)
            pltpu.async_copy(g.at[idxg2.at[t2]], rows0, sem0)
            wait_rows(rows1, sem1)
            pltpu.sync_copy(rows1, acc_sh.at[idxs2.at[t0]], add=True)
            return 0

        lax.fori_loop(0, BLK, body, 0)
        wait_rows(rows0, sem0)
        return 0

    lax.fori_loop(0, NBLK, blk_body, 0)
    plsc.subcore_barrier()
    pltpu.sync_copy(acc_sh.at[pl.ds(s * 624, 624)],
                    out.at[pl.ds(cN + s * 624, 624)])
    pltpu.sync_copy(acc_sh.at[pl.ds(t_off, 8)],
                    out.at[pl.ds(cN + t_off, 8)])


_prop_call = pl.kernel(
    _prop_body,
    out_type=jax.ShapeDtypeStruct((NC * N, DIM), jnp.float32),
    mesh=_mesh,
    scratch_types=[
        pltpu.VMEM((BLK * 2 * CHUNK,), jnp.int32),
        pltpu.VMEM((2 * BLK, CHUNK), jnp.int32),
        pltpu.VMEM((2 * BLK, CHUNK), jnp.int32),
        pltpu.VMEM((CHUNK, DIM), jnp.float32),
        pltpu.VMEM((CHUNK, DIM), jnp.float32),
        pltpu.VMEM_SHARED((NACC, DIM), jnp.float32),
        pltpu.SemaphoreType.DMA,
        pltpu.SemaphoreType.DMA,
    ],
    compiler_params=_sc_params,
)


def _seed_body(hid, seeds, out, idx_v, rows_v, sem):
    c = lax.axis_index("c")
    s = lax.axis_index("s")

    def do_chunk(j):
        pltpu.sync_copy(seeds.at[pl.ds((c * SROW + j) * CHUNK, CHUNK)], idx_v)
        pltpu.async_copy(hid.at[idx_v], rows_v, sem).wait()
        pltpu.sync_copy(rows_v, out.at[pl.ds((c * SROW + j) * CHUNK, CHUNK)])

    do_chunk(s)
    do_chunk(s + NS)
    do_chunk(jnp.minimum(s + 2 * NS, SROW - 1))


_seed_call = pl.kernel(
    _seed_body,
    out_type=jax.ShapeDtypeStruct((NC * SPAD, DIM), jnp.float32),
    mesh=_mesh,
    scratch_types=[
        pltpu.VMEM((CHUNK,), jnp.int32),
        pltpu.VMEM((CHUNK, DIM), jnp.float32),
        pltpu.SemaphoreType.DMA,
    ],
    compiler_params=_sc_params,
)


def _dense0_body(part_ref, feats_sr_ref, feats_tg_ref,
                 g0_ref, f_sr_ref, f_tg_ref):
    part = part_ref[...]
    f_sr = lax.rsqrt(jnp.sum(part[0], axis=0)[:N] + 1.0)
    f_tg = lax.rsqrt(jnp.sum(part[1], axis=0)[:N] + 1.0)
    f_sr_ref[...] = f_sr
    f_tg_ref[...] = f_tg
    g0_ref[:N, :] = feats_sr_ref[...] * f_sr[:, None]
    g0_ref[N:2 * N, :] = feats_tg_ref[...] * f_tg[:, None]


def _dense0(part, feats_sr, feats_tg):
    return pl.pallas_call(
        _dense0_body,
        out_shape=(jax.ShapeDtypeStruct((NG, DIM), jnp.float32),
                   jax.ShapeDtypeStruct((N,), jnp.float32),
                   jax.ShapeDtypeStruct((N,), jnp.float32)),
    )(part, feats_sr, feats_tg)


def _dense1_body(agg_ref, f_sr_ref, f_tg_ref, w_ref, g1_ref):
    w = w_ref[...]

    def one(agg, f):
        h = jnp.dot(agg * f[:, None], w, preferred_element_type=jnp.float32)
        return jnp.maximum(h, 0.0) * f[:, None]

    g1_ref[:N, :] = one(agg_ref[:N, :], f_sr_ref[...])
    g1_ref[N:2 * N, :] = one(agg_ref[N:, :], f_tg_ref[...])


def _dense1(agg, f_sr, f_tg, w):
    return pl.pallas_call(
        _dense1_body,
        out_shape=jax.ShapeDtypeStruct((NG, DIM), jnp.float32),
    )(agg, f_sr, f_tg, w)


def _dense2_body(agg_ref, f_sr_ref, f_tg_ref, w_ref, hid_ref):
    w = w_ref[...]

    def one(agg, f):
        h = jnp.dot(agg * f[:, None], w, preferred_element_type=jnp.float32)
        nrm = jnp.sqrt(jnp.sum(h * h, axis=-1, keepdims=True))
        return h / jnp.maximum(nrm, 1e-12)

    hid_ref[:N, :] = one(agg_ref[:N, :], f_sr_ref[...])
    hid_ref[N:, :] = one(agg_ref[N:, :], f_tg_ref[...])


def _dense2(agg, f_sr, f_tg, w):
    return pl.pallas_call(
        _dense2_body,
        out_shape=jax.ShapeDtypeStruct((NC * N, DIM), jnp.float32),
    )(agg, f_sr, f_tg, w)


def _pad_pairs(edges):
    i = jnp.arange(PADP, dtype=jnp.int32)
    padvals = N + (i % NSAC)
    pad2 = jnp.stack([padvals, padvals], axis=1).reshape(-1)
    return jnp.concatenate([edges.reshape(-1), pad2])


def _pad_seeds(seeds, node_off):
    return jnp.concatenate(
        [seeds + node_off, jnp.full((SPAD - SEEDS,), node_off, jnp.int32)])


def kernel(feats_sr, feats_tg, W0, W1, edges_sr, edges_tg,
           sr_ent_seeds, tg_ent_seeds, triples_sr, triples_tg):
    pairs = jnp.concatenate([_pad_pairs(edges_sr), _pad_pairs(edges_tg)])
    part = _deg_call(pairs)
    g0, f_sr, f_tg = _dense0(part, feats_sr, feats_tg)

    agg0 = _prop_call(g0, pairs)
    g1 = _dense1(agg0, f_sr, f_tg, W0)
    agg1 = _prop_call(g1, pairs)
    hid = _dense2(agg1, f_sr, f_tg, W1)

    seeds = jnp.concatenate(
        [_pad_seeds(sr_ent_seeds, 0), _pad_seeds(tg_ent_seeds, N)])
    seed_out = _seed_call(hid, seeds)
    return (seed_out[:SEEDS], seed_out[SPAD:SPAD + SEEDS],
            hid[:N], hid[N:])


# speedup vs baseline: 1.2288x; 1.2251x over previous
import jax
import jax.numpy as jnp
from jax import lax
from jax.experimental import pallas as pl
from jax.experimental.pallas import tpu as pltpu
from jax.experimental.pallas import tpu_sc as plsc

N = 10000
DIM = 128
E = 320000
TWO_E = 2 * E
NC, NS, L = 2, 16, 16
CHUNK = 128
BLK = 32
NCH_W = 320
EPAD = NS * NCH_W * CHUNK
NSAC = 128
NACC = N + NSAC
DEG_W = EPAD // NS
SPAD = 4608
SROW = SPAD // CHUNK
SEEDS = 4500

_mesh = plsc.VectorSubcoreMesh(
    core_axis_name="c", subcore_axis_name="s", num_cores=NC, num_subcores=NS)
_sc_params = pltpu.CompilerParams(needs_layout_passes=False)


def _deg_body(dsts, out, idx_v, hist_v):
    c = lax.axis_index("c")
    s = lax.axis_index("s")

    zeros = jnp.zeros((L,), jnp.float32)

    def zero_body(i, _):
        hist_v[pl.ds(i * L, L)] = zeros
        return 0

    lax.fori_loop(0, NACC // L, zero_body, 0)

    pltpu.sync_copy(dsts.at[pl.ds(c * EPAD + s * DEG_W, DEG_W)], idx_v)

    ones = jnp.ones((L,), jnp.float32)

    def body(i, _):
        v = idx_v[pl.ds(i * L, L)]
        plsc.addupdate_scatter(hist_v, [v], ones)
        return 0

    lax.fori_loop(0, DEG_W // L, body, 0)
    pltpu.sync_copy(hist_v, out.at[c, s])


_deg_call = pl.kernel(
    _deg_body,
    out_type=jax.ShapeDtypeStruct((NC, NS, NACC), jnp.float32),
    mesh=_mesh,
    scratch_types=[
        pltpu.VMEM((DEG_W,), jnp.int32),
        pltpu.VMEM((NACC,), jnp.float32),
    ],
    compiler_params=_sc_params,
)


def _prop_body(g, srcs, dsts, out,
               sbuf, dbuf, idxd2, rows0, rows1, acc_sh, sem0, sem1):
    c = lax.axis_index("c")
    s = lax.axis_index("s")
    cN = c * N

    t_off = 16 * 624 + 8 * jnp.minimum(s, 1)
    pltpu.sync_copy(g.at[pl.ds(cN + s * 624, 624)],
                    acc_sh.at[pl.ds(s * 624, 624)])
    pltpu.sync_copy(g.at[pl.ds(cN + t_off, 8)],
                    acc_sh.at[pl.ds(t_off, 8)])
    plsc.subcore_barrier()

    def wait_rows(buf, sem):
        pltpu.make_async_copy(g.at[pl.ds(0, CHUNK)], buf, sem).wait()

    def blk_body(b, _):
        eoff = (c * EPAD + s * NCH_W * CHUNK) + b * (BLK * CHUNK)
        pltpu.sync_copy(srcs.at[pl.ds(eoff, BLK * CHUNK)], sbuf)
        pltpu.sync_copy(dsts.at[pl.ds(eoff, BLK * CHUNK)], dbuf)

        def dcopy(i, _):
            v = dbuf[pl.ds(i * L, L)]
            idxd2[i // (CHUNK // L), pl.ds((i % (CHUNK // L)) * L, L)] = v
            return 0

        lax.fori_loop(0, BLK * CHUNK // L, dcopy, 0)

        pltpu.async_copy(g.at[sbuf.at[pl.ds(0, CHUNK)]], rows0, sem0)

        def body(i, _):
            t0 = 2 * i
            pltpu.async_copy(
                g.at[sbuf.at[pl.ds((t0 + 1) * CHUNK, CHUNK)]], rows1, sem1)
            wait_rows(rows0, sem0)
            pltpu.sync_copy(rows0, acc_sh.at[idxd2.at[t0]], add=True)
            t2 = jnp.minimum(t0 + 2, BLK - 1)
            pltpu.async_copy(
                g.at[sbuf.at[pl.ds(t2 * CHUNK, CHUNK)]], rows0, sem0)
            wait_rows(rows1, sem1)
            pltpu.sync_copy(rows1, acc_sh.at[idxd2.at[t0 + 1]], add=True)
            return 0

        lax.fori_loop(0, BLK // 2, body, 0)
        wait_rows(rows0, sem0)
        return 0

    lax.fori_loop(0, NCH_W // BLK, blk_body, 0)
    plsc.subcore_barrier()
    pltpu.sync_copy(acc_sh.at[pl.ds(s * 624, 624)],
                    out.at[pl.ds(cN + s * 624, 624)])
    pltpu.sync_copy(acc_sh.at[pl.ds(t_off, 8)],
                    out.at[pl.ds(cN + t_off, 8)])


_prop_call = pl.kernel(
    _prop_body,
    out_type=jax.ShapeDtypeStruct((NC * N, DIM), jnp.float32),
    mesh=_mesh,
    scratch_types=[
        pltpu.VMEM((BLK * CHUNK,), jnp.int32),
        pltpu.VMEM((BLK * CHUNK,), jnp.int32),
        pltpu.VMEM((BLK, CHUNK), jnp.int32),
        pltpu.VMEM((CHUNK, DIM), jnp.float32),
        pltpu.VMEM((CHUNK, DIM), jnp.float32),
        pltpu.VMEM_SHARED((NACC, DIM), jnp.float32),
        pltpu.SemaphoreType.DMA,
        pltpu.SemaphoreType.DMA,
    ],
    compiler_params=_sc_params,
)


def _seed_body(hid, seeds, out, idx_v, rows_v, sem):
    c = lax.axis_index("c")
    s = lax.axis_index("s")

    def do_chunk(j):
        pltpu.sync_copy(seeds.at[pl.ds((c * SROW + j) * CHUNK, CHUNK)], idx_v)
        pltpu.async_copy(hid.at[idx_v], rows_v, sem).wait()
        pltpu.sync_copy(rows_v, out.at[pl.ds((c * SROW + j) * CHUNK, CHUNK)])

    do_chunk(s)
    do_chunk(s + NS)
    do_chunk(jnp.minimum(s + 2 * NS, SROW - 1))


_seed_call = pl.kernel(
    _seed_body,
    out_type=jax.ShapeDtypeStruct((NC * SPAD, DIM), jnp.float32),
    mesh=_mesh,
    scratch_types=[
        pltpu.VMEM((CHUNK,), jnp.int32),
        pltpu.VMEM((CHUNK, DIM), jnp.float32),
        pltpu.SemaphoreType.DMA,
    ],
    compiler_params=_sc_params,
)


def _dense0_body(part_ref, feats_sr_ref, feats_tg_ref,
                 g0_ref, f_sr_ref, f_tg_ref):
    part = part_ref[...]
    f_sr = lax.rsqrt(jnp.sum(part[0], axis=0)[:N] + 1.0)
    f_tg = lax.rsqrt(jnp.sum(part[1], axis=0)[:N] + 1.0)
    f_sr_ref[...] = f_sr
    f_tg_ref[...] = f_tg
    g0_ref[:N, :] = feats_sr_ref[...] * f_sr[:, None]
    g0_ref[N:, :] = feats_tg_ref[...] * f_tg[:, None]


def _dense0(part, feats_sr, feats_tg):
    return pl.pallas_call(
        _dense0_body,
        out_shape=(jax.ShapeDtypeStruct((NC * N, DIM), jnp.float32),
                   jax.ShapeDtypeStruct((N,), jnp.float32),
                   jax.ShapeDtypeStruct((N,), jnp.float32)),
    )(part, feats_sr, feats_tg)


def _dense1_body(agg_ref, f_sr_ref, f_tg_ref, w_ref, g1_ref):
    w = w_ref[...]

    def one(agg, f):
        h = jnp.dot(agg * f[:, None], w, preferred_element_type=jnp.float32)
        return jnp.maximum(h, 0.0) * f[:, None]

    g1_ref[:N, :] = one(agg_ref[:N, :], f_sr_ref[...])
    g1_ref[N:, :] = one(agg_ref[N:, :], f_tg_ref[...])


def _dense1(agg, f_sr, f_tg, w):
    return pl.pallas_call(
        _dense1_body,
        out_shape=jax.ShapeDtypeStruct((NC * N, DIM), jnp.float32),
    )(agg, f_sr, f_tg, w)


def _dense2_body(agg_ref, f_sr_ref, f_tg_ref, w_ref, hid_ref):
    w = w_ref[...]

    def one(agg, f):
        h = jnp.dot(agg * f[:, None], w, preferred_element_type=jnp.float32)
        nrm = jnp.sqrt(jnp.sum(h * h, axis=-1, keepdims=True))
        return h / jnp.maximum(nrm, 1e-12)

    hid_ref[:N, :] = one(agg_ref[:N, :], f_sr_ref[...])
    hid_ref[N:, :] = one(agg_ref[N:, :], f_tg_ref[...])


def _dense2(agg, f_sr, f_tg, w):
    return pl.pallas_call(
        _dense2_body,
        out_shape=jax.ShapeDtypeStruct((NC * N, DIM), jnp.float32),
    )(agg, f_sr, f_tg, w)


def _edge_lists(edges, node_off):
    pad = EPAD - TWO_E
    spread = jnp.arange(pad, dtype=jnp.int32)
    src = jnp.concatenate(
        [edges[:, 0] + node_off, edges[:, 1] + node_off,
         node_off + (spread % N)])
    dst = jnp.concatenate(
        [edges[:, 1], edges[:, 0], N + (spread % NSAC)])
    return src, dst


def _pad_seeds(seeds, node_off):
    return jnp.concatenate(
        [seeds + node_off, jnp.full((SPAD - SEEDS,), node_off, jnp.int32)])


def kernel(feats_sr, feats_tg, W0, W1, edges_sr, edges_tg,
           sr_ent_seeds, tg_ent_seeds, triples_sr, triples_tg):
    src_sr, dst_sr = _edge_lists(edges_sr, 0)
    src_tg, dst_tg = _edge_lists(edges_tg, N)
    srcs = jnp.concatenate([src_sr, src_tg])
    dsts = jnp.concatenate([dst_sr, dst_tg])

    part = _deg_call(dsts)
    g0, f_sr, f_tg = _dense0(part, feats_sr, feats_tg)

    agg0 = _prop_call(g0, srcs, dsts)
    g1 = _dense1(agg0, f_sr, f_tg, W0)
    agg1 = _prop_call(g1, srcs, dsts)
    hid = _dense2(agg1, f_sr, f_tg, W1)

    seeds = jnp.concatenate(
        [_pad_seeds(sr_ent_seeds, 0), _pad_seeds(tg_ent_seeds, N)])
    seed_out = _seed_call(hid, seeds)
    return (seed_out[:SEEDS], seed_out[SPAD:SPAD + SEEDS],
            hid[:N], hid[N:])


# speedup vs baseline: 1.2342x; 1.0044x over previous
import jax
import jax.numpy as jnp
from jax import lax
from jax.experimental import pallas as pl
from jax.experimental.pallas import tpu as pltpu
from jax.experimental.pallas import tpu_sc as plsc

N = 10000
DIM = 128
E = 320000
TWO_E = 2 * E
NC, NS, L = 2, 16, 16
CHUNK = 128
BLK = 32
NCH_W = 320
EPAD = NS * NCH_W * CHUNK
NSAC = 128
NACC = N + NSAC
DEG_W = EPAD // NS
SPAD = 4608
SROW = SPAD // CHUNK
SEEDS = 4500

_mesh = plsc.VectorSubcoreMesh(
    core_axis_name="c", subcore_axis_name="s", num_cores=NC, num_subcores=NS)
_sc_params = pltpu.CompilerParams(needs_layout_passes=False)


def _deg_body(dsts, out, idx_v, hist_v):
    c = lax.axis_index("c")
    s = lax.axis_index("s")

    zeros = jnp.zeros((L,), jnp.float32)

    def zero_body(i, _):
        hist_v[pl.ds(i * L, L)] = zeros
        return 0

    lax.fori_loop(0, NACC // L, zero_body, 0)

    pltpu.sync_copy(dsts.at[pl.ds(c * EPAD + s * DEG_W, DEG_W)], idx_v)

    ones = jnp.ones((L,), jnp.float32)

    def body(i, _):
        v = idx_v[pl.ds(i * L, L)]
        plsc.addupdate_scatter(hist_v, [v], ones)
        return 0

    lax.fori_loop(0, DEG_W // L, body, 0)
    pltpu.sync_copy(hist_v, out.at[c, s])


_deg_call = pl.kernel(
    _deg_body,
    out_type=jax.ShapeDtypeStruct((NC, NS, NACC), jnp.float32),
    mesh=_mesh,
    scratch_types=[
        pltpu.VMEM((DEG_W,), jnp.int32),
        pltpu.VMEM((NACC,), jnp.float32),
    ],
    compiler_params=_sc_params,
)


def _prop_body(g, srcs, dsts, out,
               sbuf, dbuf, idxd2, rows0, rows1, acc_sh, sem0, sem1):
    c = lax.axis_index("c")
    s = lax.axis_index("s")
    cN = c * N

    t_off = 16 * 624 + 8 * jnp.minimum(s, 1)
    pltpu.sync_copy(g.at[pl.ds(cN + s * 624, 624)],
                    acc_sh.at[pl.ds(s * 624, 624)])
    pltpu.sync_copy(g.at[pl.ds(cN + t_off, 8)],
                    acc_sh.at[pl.ds(t_off, 8)])
    plsc.subcore_barrier()

    def wait_rows(buf, sem):
        pltpu.make_async_copy(g.at[pl.ds(0, CHUNK)], buf, sem).wait()

    def blk_body(b, _):
        eoff = (c * EPAD + s * NCH_W * CHUNK) + b * (BLK * CHUNK)
        pltpu.sync_copy(srcs.at[pl.ds(eoff, BLK * CHUNK)], sbuf)
        pltpu.sync_copy(dsts.at[pl.ds(eoff, BLK * CHUNK)], dbuf)

        def dcopy(i, _):
            v = dbuf[pl.ds(i * L, L)]
            idxd2[i // (CHUNK // L), pl.ds((i % (CHUNK // L)) * L, L)] = v
            return 0

        lax.fori_loop(0, BLK * CHUNK // L, dcopy, 0)

        pltpu.async_copy(g.at[sbuf.at[pl.ds(0, CHUNK)]], rows0, sem0)

        def body(i, _):
            t0 = 2 * i
            pltpu.async_copy(
                g.at[sbuf.at[pl.ds((t0 + 1) * CHUNK, CHUNK)]], rows1, sem1)
            wait_rows(rows0, sem0)
            pltpu.sync_copy(rows0, acc_sh.at[idxd2.at[t0]], add=True)
            t2 = jnp.minimum(t0 + 2, BLK - 1)
            pltpu.async_copy(
                g.at[sbuf.at[pl.ds(t2 * CHUNK, CHUNK)]], rows0, sem0)
            wait_rows(rows1, sem1)
            pltpu.sync_copy(rows1, acc_sh.at[idxd2.at[t0 + 1]], add=True)
            return 0

        lax.fori_loop(0, BLK // 2, body, 0)
        wait_rows(rows0, sem0)
        return 0

    lax.fori_loop(0, NCH_W // BLK, blk_body, 0)
    plsc.subcore_barrier()
    pltpu.sync_copy(acc_sh.at[pl.ds(s * 624, 624)],
                    out.at[pl.ds(cN + s * 624, 624)])
    pltpu.sync_copy(acc_sh.at[pl.ds(t_off, 8)],
                    out.at[pl.ds(cN + t_off, 8)])


_prop_call = pl.kernel(
    _prop_body,
    out_type=jax.ShapeDtypeStruct((NC * N, DIM), jnp.float32),
    mesh=_mesh,
    scratch_types=[
        pltpu.VMEM((BLK * CHUNK,), jnp.int32),
        pltpu.VMEM((BLK * CHUNK,), jnp.int32),
        pltpu.VMEM((BLK, CHUNK), jnp.int32),
        pltpu.VMEM((CHUNK, DIM), jnp.float32),
        pltpu.VMEM((CHUNK, DIM), jnp.float32),
        pltpu.VMEM_SHARED((NACC, DIM), jnp.float32),
        pltpu.SemaphoreType.DMA,
        pltpu.SemaphoreType.DMA,
    ],
    compiler_params=_sc_params,
)


def _seed_body(hid, seeds, out, i0, i1, i2, r0, r1, r2, s0, s1, s2):
    c = lax.axis_index("c")
    s = lax.axis_index("s")

    js = (s, s + NS, jnp.minimum(s + 2 * NS, SROW - 1))
    bufs = ((i0, r0, s0), (i1, r1, s1), (i2, r2, s2))
    for j, (iv, rv, sm) in zip(js, bufs):
        pltpu.sync_copy(seeds.at[pl.ds((c * SROW + j) * CHUNK, CHUNK)], iv)
        pltpu.async_copy(hid.at[iv], rv, sm)
    for j, (iv, rv, sm) in zip(js, bufs):
        pltpu.make_async_copy(hid.at[pl.ds(0, CHUNK)], rv, sm).wait()
        pltpu.sync_copy(rv, out.at[pl.ds((c * SROW + j) * CHUNK, CHUNK)])


_seed_call = pl.kernel(
    _seed_body,
    out_type=jax.ShapeDtypeStruct((NC * SPAD, DIM), jnp.float32),
    mesh=_mesh,
    scratch_types=[
        pltpu.VMEM((CHUNK,), jnp.int32),
        pltpu.VMEM((CHUNK,), jnp.int32),
        pltpu.VMEM((CHUNK,), jnp.int32),
        pltpu.VMEM((CHUNK, DIM), jnp.float32),
        pltpu.VMEM((CHUNK, DIM), jnp.float32),
        pltpu.VMEM((CHUNK, DIM), jnp.float32),
        pltpu.SemaphoreType.DMA,
        pltpu.SemaphoreType.DMA,
        pltpu.SemaphoreType.DMA,
    ],
    compiler_params=_sc_params,
)


def _dense0_body(part_ref, feats_sr_ref, feats_tg_ref,
                 g0_ref, f_sr_ref, f_tg_ref):
    part = part_ref[...]
    f_sr = lax.rsqrt(jnp.sum(part[0], axis=0)[:N] + 1.0)
    f_tg = lax.rsqrt(jnp.sum(part[1], axis=0)[:N] + 1.0)
    f_sr_ref[...] = f_sr
    f_tg_ref[...] = f_tg
    g0_ref[:N, :] = feats_sr_ref[...] * f_sr[:, None]
    g0_ref[N:, :] = feats_tg_ref[...] * f_tg[:, None]


def _dense0(part, feats_sr, feats_tg):
    return pl.pallas_call(
        _dense0_body,
        out_shape=(jax.ShapeDtypeStruct((NC * N, DIM), jnp.float32),
                   jax.ShapeDtypeStruct((N,), jnp.float32),
                   jax.ShapeDtypeStruct((N,), jnp.float32)),
    )(part, feats_sr, feats_tg)


def _dense1_body(agg_ref, f_sr_ref, f_tg_ref, w_ref, g1_ref):
    w = w_ref[...]

    def one(agg, f):
        h = jnp.dot(agg * f[:, None], w, preferred_element_type=jnp.float32)
        return jnp.maximum(h, 0.0) * f[:, None]

    g1_ref[:N, :] = one(agg_ref[:N, :], f_sr_ref[...])
    g1_ref[N:, :] = one(agg_ref[N:, :], f_tg_ref[...])


def _dense1(agg, f_sr, f_tg, w):
    return pl.pallas_call(
        _dense1_body,
        out_shape=jax.ShapeDtypeStruct((NC * N, DIM), jnp.float32),
    )(agg, f_sr, f_tg, w)


def _dense2_body(agg_ref, f_sr_ref, f_tg_ref, w_ref, hid_ref):
    w = w_ref[...]

    def one(agg, f):
        h = jnp.dot(agg * f[:, None], w, preferred_element_type=jnp.float32)
        nrm = jnp.sqrt(jnp.sum(h * h, axis=-1, keepdims=True))
        return h / jnp.maximum(nrm, 1e-12)

    hid_ref[:N, :] = one(agg_ref[:N, :], f_sr_ref[...])
    hid_ref[N:, :] = one(agg_ref[N:, :], f_tg_ref[...])


def _dense2(agg, f_sr, f_tg, w):
    return pl.pallas_call(
        _dense2_body,
        out_shape=jax.ShapeDtypeStruct((NC * N, DIM), jnp.float32),
    )(agg, f_sr, f_tg, w)


def _edge_lists(edges, node_off):
    pad = EPAD - TWO_E
    spread = jnp.arange(pad, dtype=jnp.int32)
    src = jnp.concatenate(
        [edges[:, 0] + node_off, edges[:, 1] + node_off,
         node_off + (spread % N)])
    dst = jnp.concatenate(
        [edges[:, 1], edges[:, 0], N + (spread % NSAC)])
    return src, dst


def _pad_seeds(seeds, node_off):
    return jnp.concatenate(
        [seeds + node_off, jnp.full((SPAD - SEEDS,), node_off, jnp.int32)])


def kernel(feats_sr, feats_tg, W0, W1, edges_sr, edges_tg,
           sr_ent_seeds, tg_ent_seeds, triples_sr, triples_tg):
    src_sr, dst_sr = _edge_lists(edges_sr, 0)
    src_tg, dst_tg = _edge_lists(edges_tg, N)
    srcs = jnp.concatenate([src_sr, src_tg])
    dsts = jnp.concatenate([dst_sr, dst_tg])

    part = _deg_call(dsts)
    g0, f_sr, f_tg = _dense0(part, feats_sr, feats_tg)

    agg0 = _prop_call(g0, srcs, dsts)
    g1 = _dense1(agg0, f_sr, f_tg, W0)
    agg1 = _prop_call(g1, srcs, dsts)
    hid = _dense2(agg1, f_sr, f_tg, W1)

    seeds = jnp.concatenate(
        [_pad_seeds(sr_ent_seeds, 0), _pad_seeds(tg_ent_seeds, N)])
    seed_out = _seed_call(hid, seeds)
    return (seed_out[:SEEDS], seed_out[SPAD:SPAD + SEEDS],
            hid[:N], hid[N:])
